# baseline (speedup 1.0000x reference)
import jax
import jax.numpy as jnp
from jax import lax
from jax.experimental import pallas as pl

HIDDEN_DIM = 2048
NUM_EXPERTS = 16
N_TOKENS = 16384
BLK = 2048
HALF = BLK // 2


def _router_tail(logits):
    m = jnp.max(logits, axis=1, keepdims=True)
    iota = lax.broadcasted_iota(jnp.int32, logits.shape, 1)
    i1 = jnp.min(jnp.where(logits == m, iota, NUM_EXPERTS), axis=1, keepdims=True)
    lm = jnp.where(iota == i1, -jnp.inf, logits)
    v2 = jnp.max(lm, axis=1, keepdims=True)
    i2 = jnp.min(jnp.where(lm == v2, iota, NUM_EXPERTS), axis=1, keepdims=True)

    e = jnp.exp(logits - m)
    s = jnp.sum(e, axis=1, keepdims=True)
    g1 = 1.0 / s
    g2 = jnp.exp(v2 - m) / s
    denom = g1 + g2 + 1e-9
    gtop = jnp.where(iota == i1, g1 / denom,
                     jnp.where(iota == i2, g2 / denom, 0.0))
    return gtop, jnp.concatenate([i1, i2], axis=1)


def _router_block(xa_ref, xb_ref, w_ref, b_ref, gtop_ref, idx_ref, logits_ref):
    w = w_ref[...]
    dims = (((1,), (1,)), ((), ()))
    la = lax.dot_general(xa_ref[...], w, dims,
                         preferred_element_type=jnp.float32) + b_ref[...]
    lb = lax.dot_general(xb_ref[...], w, dims,
                         preferred_element_type=jnp.float32) + b_ref[...]
    logits = jnp.concatenate([la, lb], axis=0)
    logits_ref[...] = logits
    gtop, idx = _router_tail(logits)
    gtop_ref[...] = gtop
    idx_ref[...] = idx


def kernel(x, W, b):
    n_tokens = x.shape[0]
    grid = (n_tokens // BLK,)
    g_top, idx, logits = pl.pallas_call(
        _router_block,
        grid=grid,
        in_specs=[
            pl.BlockSpec((HALF, HIDDEN_DIM), lambda i: (2 * i, 0)),
            pl.BlockSpec((HALF, HIDDEN_DIM), lambda i: (2 * i + 1, 0)),
            pl.BlockSpec((NUM_EXPERTS, HIDDEN_DIM), lambda i: (0, 0)),
            pl.BlockSpec((1, NUM_EXPERTS), lambda i: (0, 0)),
        ],
        out_specs=[
            pl.BlockSpec((BLK, NUM_EXPERTS), lambda i: (i, 0)),
            pl.BlockSpec((BLK, 2), lambda i: (i, 0)),
            pl.BlockSpec((BLK, NUM_EXPERTS), lambda i: (i, 0)),
        ],
        out_shape=[
            jax.ShapeDtypeStruct((n_tokens, NUM_EXPERTS), jnp.float32),
            jax.ShapeDtypeStruct((n_tokens, 2), jnp.int32),
            jax.ShapeDtypeStruct((n_tokens, NUM_EXPERTS), jnp.float32),
        ],
    )(x, x, W, b.reshape(1, NUM_EXPERTS))
    return (g_top, idx, logits)


# speedup vs baseline: 1.0095x; 1.0095x over previous
import jax
import jax.numpy as jnp
from jax import lax
from jax.experimental import pallas as pl

HIDDEN_DIM = 2048
NUM_EXPERTS = 16
N_TOKENS = 16384
BLK = 2048


def _router_block(x_ref, w_ref, b_ref, gtop_ref, idx_ref, logits_ref):
    logits = lax.dot_general(
        x_ref[...], w_ref[...],
        (((1,), (1,)), ((), ())),
        preferred_element_type=jnp.float32,
    ) + b_ref[...]
    logits_ref[...] = logits

    m = jnp.max(logits, axis=1, keepdims=True)
    iota = lax.broadcasted_iota(jnp.int32, logits.shape, 1)
    i1 = jnp.min(jnp.where(logits == m, iota, NUM_EXPERTS), axis=1, keepdims=True)
    lm = jnp.where(iota == i1, -jnp.inf, logits)
    v2 = jnp.max(lm, axis=1, keepdims=True)
    i2 = jnp.min(jnp.where(lm == v2, iota, NUM_EXPERTS), axis=1, keepdims=True)

    e = jnp.exp(logits - m)
    s = jnp.sum(e, axis=1, keepdims=True)
    g1 = 1.0 / s
    g2 = jnp.exp(v2 - m) / s
    denom = g1 + g2 + 1e-9
    gtop_ref[...] = jnp.where(iota == i1, g1 / denom,
                              jnp.where(iota == i2, g2 / denom, 0.0))
    idx_ref[...] = jnp.concatenate([i1, i2], axis=1)


def kernel(x, W, b):
    n_tokens = x.shape[0]
    grid = (n_tokens // BLK,)
    g_top, idx, logits = pl.pallas_call(
        _router_block,
        grid=grid,
        in_specs=[
            pl.BlockSpec((BLK, HIDDEN_DIM), lambda i: (i, 0)),
            pl.BlockSpec((NUM_EXPERTS, HIDDEN_DIM), lambda i: (0, 0)),
            pl.BlockSpec((1, NUM_EXPERTS), lambda i: (0, 0)),
        ],
        out_specs=[
            pl.BlockSpec((BLK, NUM_EXPERTS), lambda i: (i, 0)),
            pl.BlockSpec((BLK, 2), lambda i: (i, 0)),
            pl.BlockSpec((BLK, NUM_EXPERTS), lambda i: (i, 0)),
        ],
        out_shape=[
            jax.ShapeDtypeStruct((n_tokens, NUM_EXPERTS), jnp.float32),
            jax.ShapeDtypeStruct((n_tokens, 2), jnp.int32),
            jax.ShapeDtypeStruct((n_tokens, NUM_EXPERTS), jnp.float32),
        ],
    )(x, W, b.reshape(1, NUM_EXPERTS))
    return (g_top, idx, logits)
